# Initial kernel scaffold; baseline (speedup 1.0000x reference)
import jax
import jax.numpy as jnp
from jax import lax
from jax.experimental import pallas as pl
from jax.experimental.pallas import tpu as pltpu
from jax.experimental.pallas import tpu_sc as plsc

N = 10000
E = 320000
F = 16
NC, NS = 2, 16
NW = NC * NS
CHUNK = 128
EDGES_PER_W = 10240
NCHUNK = EDGES_PER_W // CHUNK
EP = NW * EDGES_PER_W
PADN = 10016
RPS = PADN // NS

_mesh = plsc.VectorSubcoreMesh(core_axis_name="c", subcore_axis_name="s")
_acc_out = jax.ShapeDtypeStruct((NC * PADN, F), jnp.float32)


@pl.kernel(
    out_type=_acc_out,
    mesh=_mesh,
    scratch_types=[
        pltpu.VMEM((CHUNK,), jnp.int32),
        pltpu.VMEM((CHUNK, F), jnp.float32),
        pltpu.VMEM_SHARED((PADN, F), jnp.float32),
    ],
)
def _deg(dst_hbm, zeros_hbm, ones_hbm, out_hbm, didx, ones_v, acc):
    cid = lax.axis_index("c")
    sid = lax.axis_index("s")
    wid = sid * NC + cid
    base = wid * EDGES_PER_W
    pltpu.sync_copy(zeros_hbm, acc.at[pl.ds(sid * RPS, RPS)])
    pltpu.sync_copy(ones_hbm, ones_v)
    plsc.subcore_barrier()

    @pl.loop(0, NCHUNK)
    def _(j):
        pltpu.sync_copy(dst_hbm.at[pl.ds(base + j * CHUNK, CHUNK)], didx)
        pltpu.sync_copy(ones_v, acc.at[didx], add=True)

    plsc.subcore_barrier()
    pltpu.sync_copy(acc.at[pl.ds(sid * RPS, RPS)],
                    out_hbm.at[pl.ds((cid * NS + sid) * RPS, RPS)])


@pl.kernel(
    out_type=_acc_out,
    mesh=_mesh,
    scratch_types=[
        pltpu.VMEM((CHUNK,), jnp.int32),
        pltpu.VMEM((CHUNK,), jnp.int32),
        pltpu.VMEM((CHUNK, F), jnp.float32),
        pltpu.VMEM_SHARED((PADN, F), jnp.float32),
    ],
)
def _agg(h_hbm, src_hbm, dst_hbm, zeros_hbm, out_hbm, sidx, didx, rows, acc):
    cid = lax.axis_index("c")
    sid = lax.axis_index("s")
    wid = sid * NC + cid
    base = wid * EDGES_PER_W
    pltpu.sync_copy(zeros_hbm, acc.at[pl.ds(sid * RPS, RPS)])
    plsc.subcore_barrier()

    @pl.loop(0, NCHUNK)
    def _(j):
        pltpu.sync_copy(src_hbm.at[pl.ds(base + j * CHUNK, CHUNK)], sidx)
        pltpu.sync_copy(dst_hbm.at[pl.ds(base + j * CHUNK, CHUNK)], didx)
        pltpu.sync_copy(h_hbm.at[sidx], rows)
        pltpu.sync_copy(rows, acc.at[didx], add=True)

    plsc.subcore_barrier()
    pltpu.sync_copy(acc.at[pl.ds(sid * RPS, RPS)],
                    out_hbm.at[pl.ds((cid * NS + sid) * RPS, RPS)])


def _tc1_body(x_ref, w1_ref, degp_ref, h1s_ref, dinv_ref):
    deg = degp_ref[0:N, 0:1] + degp_ref[PADN:PADN + N, 0:1] + 1.0
    dinv = lax.rsqrt(deg)
    h1 = jnp.dot(x_ref[...], w1_ref[...], preferred_element_type=jnp.float32,
                 precision=lax.Precision.HIGHEST)
    h1s_ref[...] = h1 * dinv
    dinv_ref[...] = dinv


_tc1 = pl.pallas_call(
    _tc1_body,
    out_shape=(jax.ShapeDtypeStruct((N, F), jnp.float32),
               jax.ShapeDtypeStruct((N, 1), jnp.float32)),
)


def _tc2_body(sp_ref, h1s_ref, dinv_ref, w2_ref, b1_ref, h2s_ref):
    s = sp_ref[0:N, :] + sp_ref[PADN:PADN + N, :]
    dinv = dinv_ref[...]
    z1 = jnp.maximum((s + h1s_ref[...]) * dinv + b1_ref[...], 0.0)
    h2 = jnp.dot(z1, w2_ref[...], preferred_element_type=jnp.float32,
                 precision=lax.Precision.HIGHEST)
    h2s_ref[...] = h2 * dinv


_tc2 = pl.pallas_call(
    _tc2_body,
    out_shape=jax.ShapeDtypeStruct((N, F), jnp.float32),
)


def _tc3_body(sp_ref, h2s_ref, dinv_ref, b2_ref, out_ref):
    s = sp_ref[0:N, :] + sp_ref[PADN:PADN + N, :]
    out_ref[...] = (s + h2s_ref[...]) * dinv_ref[...] + b2_ref[...]


_tc3 = pl.pallas_call(
    _tc3_body,
    out_shape=jax.ShapeDtypeStruct((N, F), jnp.float32),
)


def kernel(x, edge_index, W1, b1, W2, b2):
    src = edge_index[0].astype(jnp.int32)
    dst = edge_index[1].astype(jnp.int32)
    pad = EP - E
    src_p = jnp.concatenate([src, jnp.zeros((pad,), jnp.int32)])
    dst_p = jnp.concatenate([dst, jnp.full((pad,), N, jnp.int32)])
    zeros = jnp.zeros((RPS, F), jnp.float32)
    ones = jnp.ones((CHUNK, F), jnp.float32)

    degp = _deg(dst_p, zeros, ones)
    h1s, dinv = _tc1(x, W1, degp)
    s1p = _agg(h1s, src_p, dst_p, zeros)
    h2s = _tc2(s1p, h1s, dinv, W2, b1.reshape(1, F))
    s2p = _agg(h2s, src_p, dst_p, zeros)
    return _tc3(s2p, h2s, dinv, b2.reshape(1, F))


# speedup vs baseline: 19.1802x; 19.1802x over previous
import jax
import jax.numpy as jnp
from jax import lax
from jax.experimental import pallas as pl
from jax.experimental.pallas import tpu as pltpu
from jax.experimental.pallas import tpu_sc as plsc

N = 10000
E = 320000
F = 16
NC, NS = 2, 16
NW = NC * NS
CHUNK = 128
EDGES_PER_W = 10240
NCHUNK = EDGES_PER_W // CHUNK
EP = NW * EDGES_PER_W
PADN = 10112
RPS = PADN // NS

_mesh = plsc.VectorSubcoreMesh(core_axis_name="c", subcore_axis_name="s")
_acc_out = jax.ShapeDtypeStruct((NC * PADN, F), jnp.float32)
_sc_params = pltpu.CompilerParams(use_tc_tiling_on_sc=False)


@pl.kernel(
    out_type=_acc_out,
    mesh=_mesh,
    scratch_types=[
        pltpu.VMEM((CHUNK,), jnp.int32),
        pltpu.VMEM((CHUNK, F), jnp.float32),
        pltpu.VMEM_SHARED((PADN, F), jnp.float32),
    ],
    compiler_params=_sc_params,
)
def _deg(dst_hbm, zeros_hbm, ones_hbm, out_hbm, didx, ones_v, acc):
    cid = lax.axis_index("c")
    sid = lax.axis_index("s")
    wid = sid * NC + cid
    base = wid * EDGES_PER_W
    pltpu.sync_copy(zeros_hbm, acc.at[pl.ds(sid * RPS, RPS)])
    pltpu.sync_copy(ones_hbm, ones_v)
    plsc.subcore_barrier()

    @pl.loop(0, NCHUNK)
    def _(j):
        pltpu.sync_copy(dst_hbm.at[pl.ds(base + j * CHUNK, CHUNK)], didx)
        pltpu.sync_copy(ones_v, acc.at[didx], add=True)

    plsc.subcore_barrier()
    pltpu.sync_copy(acc.at[pl.ds(sid * RPS, RPS)],
                    out_hbm.at[pl.ds((cid * NS + sid) * RPS, RPS)])


@pl.kernel(
    out_type=_acc_out,
    mesh=_mesh,
    scratch_types=[
        pltpu.VMEM((CHUNK,), jnp.int32),
        pltpu.VMEM((CHUNK,), jnp.int32),
        pltpu.VMEM((CHUNK, F), jnp.float32),
        pltpu.VMEM_SHARED((PADN, F), jnp.float32),
    ],
    compiler_params=_sc_params,
)
def _agg(h_hbm, src_hbm, dst_hbm, zeros_hbm, out_hbm, sidx, didx, rows, acc):
    cid = lax.axis_index("c")
    sid = lax.axis_index("s")
    wid = sid * NC + cid
    base = wid * EDGES_PER_W
    pltpu.sync_copy(zeros_hbm, acc.at[pl.ds(sid * RPS, RPS)])
    plsc.subcore_barrier()

    @pl.loop(0, NCHUNK)
    def _(j):
        pltpu.sync_copy(src_hbm.at[pl.ds(base + j * CHUNK, CHUNK)], sidx)
        pltpu.sync_copy(dst_hbm.at[pl.ds(base + j * CHUNK, CHUNK)], didx)
        pltpu.sync_copy(h_hbm.at[sidx], rows)
        pltpu.sync_copy(rows, acc.at[didx], add=True)

    plsc.subcore_barrier()
    pltpu.sync_copy(acc.at[pl.ds(sid * RPS, RPS)],
                    out_hbm.at[pl.ds((cid * NS + sid) * RPS, RPS)])


def _tc1_body(x_ref, w1_ref, degp_ref, h1s_ref, dinv_ref):
    deg = degp_ref[0:N, 0:1] + degp_ref[PADN:PADN + N, 0:1] + 1.0
    dinv = lax.rsqrt(deg)
    h1 = jnp.dot(x_ref[...], w1_ref[...], preferred_element_type=jnp.float32,
                 precision=lax.Precision.HIGHEST)
    h1s_ref[...] = h1 * dinv
    dinv_ref[...] = dinv


_tc1 = pl.pallas_call(
    _tc1_body,
    out_shape=(jax.ShapeDtypeStruct((N, F), jnp.float32),
               jax.ShapeDtypeStruct((N, 1), jnp.float32)),
)


def _tc2_body(sp_ref, h1s_ref, dinv_ref, w2_ref, b1_ref, h2s_ref):
    s = sp_ref[0:N, :] + sp_ref[PADN:PADN + N, :]
    dinv = dinv_ref[...]
    z1 = jnp.maximum((s + h1s_ref[...]) * dinv + b1_ref[...], 0.0)
    h2 = jnp.dot(z1, w2_ref[...], preferred_element_type=jnp.float32,
                 precision=lax.Precision.HIGHEST)
    h2s_ref[...] = h2 * dinv


_tc2 = pl.pallas_call(
    _tc2_body,
    out_shape=jax.ShapeDtypeStruct((N, F), jnp.float32),
)


def _tc3_body(sp_ref, h2s_ref, dinv_ref, b2_ref, out_ref):
    s = sp_ref[0:N, :] + sp_ref[PADN:PADN + N, :]
    out_ref[...] = (s + h2s_ref[...]) * dinv_ref[...] + b2_ref[...]


_tc3 = pl.pallas_call(
    _tc3_body,
    out_shape=jax.ShapeDtypeStruct((N, F), jnp.float32),
)


def kernel(x, edge_index, W1, b1, W2, b2):
    src = edge_index[0].astype(jnp.int32)
    dst = edge_index[1].astype(jnp.int32)
    pad = EP - E
    src_p = jnp.concatenate([src, jnp.zeros((pad,), jnp.int32)])
    dst_p = jnp.concatenate([dst, jnp.full((pad,), N, jnp.int32)])
    zeros = jnp.zeros((RPS, F), jnp.float32)
    ones = jnp.ones((CHUNK, F), jnp.float32)

    degp = _deg(dst_p, zeros, ones)
    h1s, dinv = _tc1(x, W1, degp)
    s1p = _agg(h1s, src_p, dst_p, zeros)
    h2s = _tc2(s1p, h1s, dinv, W2, b1.reshape(1, F))
    s2p = _agg(h2s, src_p, dst_p, zeros)
    return _tc3(s2p, h2s, dinv, b2.reshape(1, F))


# speedup vs baseline: 36.0636x; 1.8803x over previous
import jax
import jax.numpy as jnp
from jax import lax
from jax.experimental import pallas as pl
from jax.experimental.pallas import tpu as pltpu
from jax.experimental.pallas import tpu_sc as plsc

N = 10000
E = 320000
F = 16
NC, NS = 2, 16
NW = NC * NS
CHUNK = 128
EDGES_PER_W = 10240
NCHUNK = EDGES_PER_W // CHUNK
EP = NW * EDGES_PER_W
PADN = 10112
RPS = PADN // NS

_mesh = plsc.VectorSubcoreMesh(core_axis_name="c", subcore_axis_name="s")
_acc_out = jax.ShapeDtypeStruct((NC * PADN, F), jnp.float32)
_sc_params = pltpu.CompilerParams(use_tc_tiling_on_sc=False)


NB = 4


@pl.kernel(
    out_type=_acc_out,
    mesh=_mesh,
    scratch_types=[
        pltpu.VMEM((NCHUNK, CHUNK), jnp.int32),
        pltpu.VMEM((CHUNK, F), jnp.float32),
        pltpu.VMEM_SHARED((PADN, F), jnp.float32),
    ] + [pltpu.SemaphoreType.DMA] * NB,
    compiler_params=_sc_params,
)
def _deg(dst_hbm, zeros_hbm, ones_hbm, out_hbm, didx, ones_v, acc, *ssems):
    cid = lax.axis_index("c")
    sid = lax.axis_index("s")
    wid = sid * NC + cid
    pltpu.sync_copy(zeros_hbm, acc.at[pl.ds(sid * RPS, RPS)])
    pltpu.sync_copy(ones_hbm, ones_v)
    pltpu.sync_copy(dst_hbm.at[wid], didx)
    plsc.subcore_barrier()

    def sdesc(jj, b):
        return pltpu.make_async_copy(ones_v, acc.at[didx.at[jj]], ssems[b])

    @pl.loop(0, NCHUNK, step=NB)
    def _(j):
        for b in range(NB):
            jj = j + b

            @pl.when(j > 0)
            def _():
                sdesc(jj - NB, b).wait()

            sdesc(jj, b).start(add=True)

    for b in range(NB):
        sdesc(NCHUNK - NB + b, b).wait()
    plsc.subcore_barrier()
    pltpu.sync_copy(acc.at[pl.ds(sid * RPS, RPS)],
                    out_hbm.at[pl.ds((cid * NS + sid) * RPS, RPS)])


@pl.kernel(
    out_type=_acc_out,
    mesh=_mesh,
    scratch_types=[
        pltpu.VMEM((EDGES_PER_W,), jnp.int32),
        pltpu.VMEM((NCHUNK, CHUNK), jnp.int32),
        [pltpu.VMEM((CHUNK, F), jnp.float32)] * NB,
        pltpu.VMEM_SHARED((PADN, F), jnp.float32),
        [pltpu.SemaphoreType.DMA] * NB,
        [pltpu.SemaphoreType.DMA] * NB,
    ],
    compiler_params=_sc_params,
)
def _agg(h_hbm, src_hbm, dst_hbm, zeros_hbm, out_hbm,
         sidx, didx, rows, acc, gsems, ssems):
    cid = lax.axis_index("c")
    sid = lax.axis_index("s")
    wid = sid * NC + cid
    pltpu.sync_copy(zeros_hbm, acc.at[pl.ds(sid * RPS, RPS)])
    pltpu.sync_copy(src_hbm.at[wid], sidx)
    pltpu.sync_copy(dst_hbm.at[wid], didx)
    plsc.subcore_barrier()

    def gdesc(jj, b):
        idx = sidx.at[pl.ds(jj * CHUNK, CHUNK)]
        return pltpu.make_async_copy(h_hbm.at[idx], rows[b], gsems[b])

    def sdesc(jj, b):
        return pltpu.make_async_copy(rows[b], acc.at[didx.at[jj]], ssems[b])

    for b in range(NB):
        gdesc(b, b).start()

    @pl.loop(0, NCHUNK, step=NB)
    def _(j):
        for b in range(NB):
            jj = j + b
            gdesc(jj, b).wait()
            sdesc(jj, b).start(add=True)
        for b in range(NB):
            jj = j + b
            sdesc(jj, b).wait()

            @pl.when(jj + NB < NCHUNK)
            def _():
                gdesc(jj + NB, b).start()

    plsc.subcore_barrier()
    pltpu.sync_copy(acc.at[pl.ds(sid * RPS, RPS)],
                    out_hbm.at[pl.ds((cid * NS + sid) * RPS, RPS)])


def _tc1_body(x_ref, w1_ref, degp_ref, h1s_ref, dinv_ref):
    deg = degp_ref[0:N, 0:1] + degp_ref[PADN:PADN + N, 0:1] + 1.0
    dinv = lax.rsqrt(deg)
    h1 = jnp.dot(x_ref[...], w1_ref[...], preferred_element_type=jnp.float32,
                 precision=lax.Precision.HIGHEST)
    h1s_ref[...] = h1 * dinv
    dinv_ref[...] = dinv


_tc1 = pl.pallas_call(
    _tc1_body,
    out_shape=(jax.ShapeDtypeStruct((N, F), jnp.float32),
               jax.ShapeDtypeStruct((N, 1), jnp.float32)),
)


def _tc2_body(sp_ref, h1s_ref, dinv_ref, w2_ref, b1_ref, h2s_ref):
    s = sp_ref[0:N, :] + sp_ref[PADN:PADN + N, :]
    dinv = dinv_ref[...]
    z1 = jnp.maximum((s + h1s_ref[...]) * dinv + b1_ref[...], 0.0)
    h2 = jnp.dot(z1, w2_ref[...], preferred_element_type=jnp.float32,
                 precision=lax.Precision.HIGHEST)
    h2s_ref[...] = h2 * dinv


_tc2 = pl.pallas_call(
    _tc2_body,
    out_shape=jax.ShapeDtypeStruct((N, F), jnp.float32),
)


def _tc3_body(sp_ref, h2s_ref, dinv_ref, b2_ref, out_ref):
    s = sp_ref[0:N, :] + sp_ref[PADN:PADN + N, :]
    out_ref[...] = (s + h2s_ref[...]) * dinv_ref[...] + b2_ref[...]


_tc3 = pl.pallas_call(
    _tc3_body,
    out_shape=jax.ShapeDtypeStruct((N, F), jnp.float32),
)


def kernel(x, edge_index, W1, b1, W2, b2):
    src = edge_index[0].astype(jnp.int32)
    dst = edge_index[1].astype(jnp.int32)
    pad = EP - E
    src_p = jnp.concatenate([src, jnp.zeros((pad,), jnp.int32)])
    dst_p = jnp.concatenate([dst, jnp.full((pad,), N, jnp.int32)])
    src_p = src_p.reshape(NW, EDGES_PER_W)
    dst_p = dst_p.reshape(NW, NCHUNK, CHUNK)
    zeros = jnp.zeros((RPS, F), jnp.float32)
    ones = jnp.ones((CHUNK, F), jnp.float32)

    degp = _deg(dst_p, zeros, ones)
    h1s, dinv = _tc1(x, W1, degp)
    s1p = _agg(h1s, src_p, dst_p, zeros)
    h2s = _tc2(s1p, h1s, dinv, W2, b1.reshape(1, F))
    s2p = _agg(h2s, src_p, dst_p, zeros)
    return _tc3(s2p, h2s, dinv, b2.reshape(1, F))


# speedup vs baseline: 63.8935x; 1.7717x over previous
import jax
import jax.numpy as jnp
from jax import lax
from jax.experimental import pallas as pl
from jax.experimental.pallas import tpu as pltpu
from jax.experimental.pallas import tpu_sc as plsc

N = 10000
E = 320000
F = 16
NC, NS = 2, 16
NW = NC * NS
EW = E // NW
CHUNK = 512
NFULL = EW // CHUNK
TAIL = EW - NFULL * CHUNK
NT = NFULL + 1
NB = 4
PADN = 10112
RPS = PADN // NS

_mesh = plsc.VectorSubcoreMesh(core_axis_name="c", subcore_axis_name="s")
_acc_out = jax.ShapeDtypeStruct((NC * PADN, F), jnp.float32)
_sc_params = pltpu.CompilerParams(use_tc_tiling_on_sc=False)


def _clen(j):
    return CHUNK if j < NFULL else TAIL


@pl.kernel(
    out_type=_acc_out,
    mesh=_mesh,
    scratch_types=[
        pltpu.VMEM((EW,), jnp.int32),
        pltpu.VMEM((CHUNK, F), jnp.float32),
        pltpu.VMEM_SHARED((PADN, F), jnp.float32),
    ] + [pltpu.SemaphoreType.DMA] * NB,
    compiler_params=_sc_params,
)
def _deg(ei_hbm, zeros_hbm, ones_hbm, out_hbm, didx, ones_v, acc, *ssems):
    cid = lax.axis_index("c")
    sid = lax.axis_index("s")
    wid = sid * NC + cid
    base = wid * EW
    pltpu.sync_copy(zeros_hbm, acc.at[pl.ds(sid * RPS, RPS)])
    pltpu.sync_copy(ones_hbm, ones_v)
    pltpu.sync_copy(ei_hbm.at[1, pl.ds(base, EW)], didx)
    plsc.subcore_barrier()

    def sd(j, b):
        ll = _clen(j)
        src = ones_v if ll == CHUNK else ones_v.at[pl.ds(0, TAIL)]
        return pltpu.make_async_copy(
            src, acc.at[didx.at[pl.ds(j * CHUNK, ll)]], ssems[b])

    for j0 in range(0, NT, NB):
        for b in range(NB):
            j = j0 + b
            if j >= NT:
                break
            if j - NB >= 0:
                sd(j - NB, b).wait()
            sd(j, b).start(add=True)
    for b in range(NB):
        j = NT - NB + b
        if j >= 0:
            sd(j, b).wait()
    plsc.subcore_barrier()
    pltpu.sync_copy(acc.at[pl.ds(sid * RPS, RPS)],
                    out_hbm.at[pl.ds((cid * NS + sid) * RPS, RPS)])


@pl.kernel(
    out_type=_acc_out,
    mesh=_mesh,
    scratch_types=[
        pltpu.VMEM((EW,), jnp.int32),
        pltpu.VMEM((EW,), jnp.int32),
        [pltpu.VMEM((CHUNK, F), jnp.float32)] * NB,
        pltpu.VMEM_SHARED((PADN, F), jnp.float32),
        [pltpu.SemaphoreType.DMA] * NB,
        [pltpu.SemaphoreType.DMA] * NB,
    ],
    compiler_params=_sc_params,
)
def _agg(h_hbm, ei_hbm, zeros_hbm, out_hbm, sidx, didx, rows, acc,
         gsems, ssems):
    cid = lax.axis_index("c")
    sid = lax.axis_index("s")
    wid = sid * NC + cid
    base = wid * EW
    pltpu.sync_copy(zeros_hbm, acc.at[pl.ds(sid * RPS, RPS)])
    pltpu.sync_copy(ei_hbm.at[0, pl.ds(base, EW)], sidx)
    pltpu.sync_copy(ei_hbm.at[1, pl.ds(base, EW)], didx)
    plsc.subcore_barrier()

    def buf(j, b):
        return rows[b] if _clen(j) == CHUNK else rows[b].at[pl.ds(0, TAIL)]

    def gd(j, b):
        idx = sidx.at[pl.ds(j * CHUNK, _clen(j))]
        return pltpu.make_async_copy(h_hbm.at[idx], buf(j, b), gsems[b])

    def sd(j, b):
        idx = didx.at[pl.ds(j * CHUNK, _clen(j))]
        return pltpu.make_async_copy(buf(j, b), acc.at[idx], ssems[b])

    for b in range(NB):
        gd(b, b).start()
    for j0 in range(0, NT, NB):
        for b in range(NB):
            j = j0 + b
            if j >= NT:
                break
            gd(j, b).wait()
            sd(j, b).start(add=True)
        for b in range(NB):
            j = j0 + b
            if j >= NT:
                break
            sd(j, b).wait()
            if j + NB < NT:
                gd(j + NB, b).start()
    plsc.subcore_barrier()
    pltpu.sync_copy(acc.at[pl.ds(sid * RPS, RPS)],
                    out_hbm.at[pl.ds((cid * NS + sid) * RPS, RPS)])


def _tc1_body(x_ref, w1_ref, degp_ref, h1s_ref, dinv_ref):
    deg = degp_ref[0:N, 0:1] + degp_ref[PADN:PADN + N, 0:1] + 1.0
    dinv = lax.rsqrt(deg)
    h1 = jnp.dot(x_ref[...], w1_ref[...], preferred_element_type=jnp.float32)
    h1s_ref[...] = h1 * dinv
    dinv_ref[...] = dinv


_tc1 = pl.pallas_call(
    _tc1_body,
    out_shape=(jax.ShapeDtypeStruct((N, F), jnp.float32),
               jax.ShapeDtypeStruct((N, 1), jnp.float32)),
)


def _tc2_body(sp_ref, h1s_ref, dinv_ref, w2_ref, b1_ref, h2s_ref):
    s = sp_ref[0:N, :] + sp_ref[PADN:PADN + N, :]
    dinv = dinv_ref[...]
    z1 = jnp.maximum((s + h1s_ref[...]) * dinv + b1_ref[...], 0.0)
    h2 = jnp.dot(z1, w2_ref[...], preferred_element_type=jnp.float32)
    h2s_ref[...] = h2 * dinv


_tc2 = pl.pallas_call(
    _tc2_body,
    out_shape=jax.ShapeDtypeStruct((N, F), jnp.float32),
)


def _tc3_body(sp_ref, h2s_ref, dinv_ref, b2_ref, out_ref):
    s = sp_ref[0:N, :] + sp_ref[PADN:PADN + N, :]
    out_ref[...] = (s + h2s_ref[...]) * dinv_ref[...] + b2_ref[...]


_tc3 = pl.pallas_call(
    _tc3_body,
    out_shape=jax.ShapeDtypeStruct((N, F), jnp.float32),
)


def kernel(x, edge_index, W1, b1, W2, b2):
    ei = edge_index.astype(jnp.int32)
    zeros = jnp.zeros((RPS, F), jnp.float32)
    ones = jnp.ones((CHUNK, F), jnp.float32)

    degp = _deg(ei, zeros, ones)
    h1s, dinv = _tc1(x, W1, degp)
    s1p = _agg(h1s, ei, zeros)
    h2s = _tc2(s1p, h1s, dinv, W2, b1.reshape(1, F))
    s2p = _agg(h2s, ei, zeros)
    return _tc3(s2p, h2s, dinv, b2.reshape(1, F))
